# Initial kernel scaffold; baseline (speedup 1.0000x reference)
import jax
import jax.numpy as jnp
from jax.experimental import pallas as pl


def kernel(x, edge_index, edge_attr, batch, Wa, ba, Wb, bb, Wn, We, Bw, Lw, Lb, Wg, bg, Fa1, Fa2, Wout, bout):
    raise NotImplementedError("write your pallas kernel here")


# speedup vs baseline: 1.2529x; 1.2529x over previous
import functools

import jax
import jax.numpy as jnp
from jax import lax
from jax.experimental import pallas as pl
from jax.experimental.pallas import tpu as pltpu
from jax.experimental.pallas import tpu_sc as plsc

N = 10000
E = 320000
H = 128
ED = 16
S = 8
G = 64
OUT = 128
L = 3

NC = 2
NS = 16
NW = NC * NS
EPW = E // NW
K = 80
C = EPW // K

BE = 1280
EBLK = E // BE


def _sc_gather(xh, src1, dst1):
    @functools.partial(
        pl.kernel,
        mesh=plsc.VectorSubcoreMesh(core_axis_name="c", subcore_axis_name="s"),
        out_type=[
            jax.ShapeDtypeStruct((E, H), jnp.float32),
            jax.ShapeDtypeStruct((E, H), jnp.float32),
        ],
        scratch_types=[
            pltpu.VMEM((K,), jnp.int32),
            pltpu.VMEM((K,), jnp.int32),
            pltpu.VMEM((K, H), jnp.float32),
            pltpu.VMEM((K, H), jnp.float32),
            pltpu.SemaphoreType.DMA,
            pltpu.SemaphoreType.DMA,
        ],
    )
    def gk(xh_hbm, src_hbm, dst_hbm, xi_hbm, xj_hbm, idxs, idxd, rs, rd, s1, s2):
        cid = lax.axis_index("c")
        sid = lax.axis_index("s")
        w = sid * NC + cid

        def body(c, carry):
            base = w * EPW + c * K
            pltpu.sync_copy(src_hbm.at[pl.ds(base, K)], idxs)
            pltpu.sync_copy(dst_hbm.at[pl.ds(base, K)], idxd)
            cp1 = pltpu.async_copy(xh_hbm.at[idxs], rs, s1)
            cp2 = pltpu.async_copy(xh_hbm.at[idxd], rd, s2)
            cp1.wait()
            cp2.wait()
            pltpu.sync_copy(rs, xj_hbm.at[pl.ds(base, K)])
            pltpu.sync_copy(rd, xi_hbm.at[pl.ds(base, K)])
            return carry

        lax.fori_loop(0, C, body, 0)

    return gk(xh, src1, dst1)


def _edge_tmp(Xi, W2):
    def body(xi_ref, w2_ref, o_ref):
        o_ref[...] = lax.dot_general(xi_ref[...], w2_ref[...],
                                     (((1,), (0,)), ((), ())))

    return pl.pallas_call(
        body,
        grid=(EBLK,),
        in_specs=[pl.BlockSpec((BE, H), lambda i: (i, 0)),
                  pl.BlockSpec((H, S * H), lambda i: (0, 0))],
        out_specs=pl.BlockSpec((BE, S * H), lambda i: (i, 0)),
        out_shape=jax.ShapeDtypeStruct((E, S * H), jnp.float32),
    )(Xi, W2)


def kernel(x, edge_index, edge_attr, batch, Wa, ba, Wb, bb, Wn, We, Bw, Lw,
           Lb, Wg, bg, Fa1, Fa2, Wout, bout):
    src = edge_index[0].astype(jnp.int32)
    dst = edge_index[1].astype(jnp.int32)

    x = jax.nn.relu(x @ Wa.T + ba)
    ea0 = jax.nn.relu(edge_attr @ Wb.T + bb)
    for l in range(L):
        xh = x @ Wn[l]
        ea = ea0 @ We[l]
        x_i, x_j = _sc_gather(xh, src, dst)
        W2 = Bw[l].transpose(1, 0, 2).reshape(H, S * H)
        tmp = _edge_tmp(x_i, W2).reshape(E, S, H)
        score = lax.dot_general(tmp, x_j, (((2,), (1,)), ((0,), (0,))))
        vec = jnp.concatenate([x_i, ea, x_j], axis=1)
        block = vec @ Lw[l].T + Lb[l]
        alpha = jnp.tanh(score + block)
        msg = (jnp.maximum(x_j, ea).reshape(-1, S, H // S)
               * alpha[:, :, None]).reshape(-1, H)
        agg = jax.ops.segment_sum(msg, dst, num_segments=N)
        h = jax.nn.relu(agg)
        beta = jax.nn.sigmoid(jnp.concatenate([x, h, x - h], axis=1) @ Wg.T
                              + bg)
        x = beta * x + (1.0 - beta) * h
        mx = jax.ops.segment_max(x, batch, num_segments=G)
        mx = jnp.where(jnp.isfinite(mx), mx, 0.0)
        sm = jax.ops.segment_sum(x, batch, num_segments=G)

        def mlp(t):
            return jax.nn.relu(t @ Fa1.T) @ Fa2.T

        y = jax.nn.sigmoid(mlp(mx) + mlp(sm))
        x = x * y[batch]
    mol = jax.nn.relu(jax.ops.segment_sum(x, batch, num_segments=G))
    return mol @ Wout.T + bout
